# baseline (speedup 1.0000x reference)
import functools

import jax
import jax.numpy as jnp
from jax import lax
from jax.experimental import pallas as pl
from jax.experimental.pallas import tpu as pltpu
from jax.experimental.pallas import tpu_sc as plsc

M = 100000
B = 16384
D = 128
NW = 32
S = 3136
S_LAST = M - S * (NW - 1)
T = S
CH = 64
WB = S + CH
NCHUNK = B // 16
DPS = 64
BF = 8


def _dedup_chunks(idx_v, table_v, row_lo, n_own, iota, base, chunks):
  for group_start in range(0, len(chunks), BF):
    group = chunks[group_start:group_start + BF]
    ivecs = [idx_v[pl.ds((base + c) * 16, 16)] for c in group]
    locals_ = [ivec - row_lo for ivec in ivecs]
    valids = [(l >= 0) & (l < n_own) for l in locals_]
    lasts = [plsc.scan_count(ivec, mask=v)[1]
             for ivec, v in zip(ivecs, valids)]
    for cc, l, v, last in zip(group, locals_, valids, lasts):
      m = v & last
      l_c = jnp.clip(l, 0, T - 1)
      plsc.store_scatter(table_v, [l_c], (base + cc) * 16 + iota, mask=m)


def _body_a(idx_hbm, val_hbm, nodes_hbm, cnt_hbm, wbuf_hbm, idx_v, table_v,
            nodes_v, gidx_v, cnt_v, rows_v0, rows_v1, wgsem, wssem0, wssem1):
  c = lax.axis_index("c")
  s = lax.axis_index("s")
  wid = s * 2 + c
  row_lo = wid * S
  n_own = jnp.where(wid == NW - 1, S_LAST, S)
  rows_vs = (rows_v0, rows_v1)
  wssems = (wssem0, wssem1)
  iota = lax.iota(jnp.int32, 16)

  pltpu.sync_copy(idx_hbm, idx_v)

  minus1 = jnp.full((16,), -1, jnp.int32)

  def zero_body(i, carry):
    for u in range(4):
      table_v[pl.ds((i * 4 + u) * 16, 16)] = minus1
    return carry

  lax.fori_loop(0, T // 16 // 4, zero_body, 0)

  def scan_body(i, carry):
    _dedup_chunks(idx_v, table_v, row_lo, n_own, iota, i * DPS,
                  list(range(DPS)))
    return carry

  lax.fori_loop(0, NCHUNK // DPS, scan_body, 0)

  def extract_body(t, off):
    tv = table_v[pl.ds(t * 16, 16)]
    m = tv >= 0
    nodes = (row_lo + t * 16) + iota
    plsc.store_compressed(nodes_v.at[pl.ds(off, 16)], nodes, mask=m)
    plsc.store_compressed(gidx_v.at[pl.ds(off, 16)], tv, mask=m)
    return off + jnp.sum(m.astype(jnp.int32))

  nwin = lax.fori_loop(0, T // 16, extract_body, jnp.int32(0))

  @pl.when(nwin > 0)
  def _():
    lane0 = (iota == 0).astype(jnp.int32)
    n0 = jnp.sum(nodes_v[pl.ds(0, 16)] * lane0)
    g0 = jnp.sum(gidx_v[pl.ds(0, 16)] * lane0)
    npad = jnp.zeros((16,), jnp.int32) + n0
    gpad = jnp.zeros((16,), jnp.int32) + g0
    for k in range(CH // 16):
      nodes_v[pl.ds(nwin + k * 16, 16)] = npad
      gidx_v[pl.ds(nwin + k * 16, 16)] = gpad

  cnt_v[...] = jnp.zeros((16,), jnp.int32) + nwin
  pltpu.sync_copy(cnt_v, cnt_hbm.at[wid])
  pltpu.sync_copy(nodes_v, nodes_hbm.at[wid])

  nchunks = (nwin + CH - 1) // CH

  def chunk_body(ci, carry):
    off = ci * CH
    for par in range(2):
      @pl.when(lax.rem(ci, 2) == par)
      def _():
        rb = rows_vs[par]

        @pl.when(ci >= 2)
        def _():
          pltpu.make_async_copy(
              rb, wbuf_hbm.at[wid].at[pl.ds(off - 2 * CH, CH)],
              wssems[par]).wait()

        pltpu.async_copy(val_hbm.at[gidx_v.at[pl.ds(off, CH)]], rb,
                         wgsem).wait()
        pltpu.make_async_copy(rb, wbuf_hbm.at[wid].at[pl.ds(off, CH)],
                              wssems[par]).start()
    return carry

  lax.fori_loop(0, nchunks, chunk_body, 0)

  @pl.when(nchunks >= 1)
  def _():
    ci = nchunks - 1
    par = lax.rem(ci, 2)
    for p in range(2):
      @pl.when(par == p)
      def _():
        pltpu.make_async_copy(
            rows_vs[p], wbuf_hbm.at[wid].at[pl.ds(ci * CH, CH)],
            wssems[p]).wait()

  @pl.when(nchunks >= 2)
  def _():
    ci = nchunks - 2
    par = lax.rem(ci, 2)
    for p in range(2):
      @pl.when(par == p)
      def _():
        pltpu.make_async_copy(
            rows_vs[p], wbuf_hbm.at[wid].at[pl.ds(ci * CH, CH)],
            wssems[p]).wait()


def _body_b(nodes_hbm, cnt_hbm, wbuf_hbm, out_hbm, cnt_v, nodes_v,
            nchunk_n0, nchunk_n1, rows_v0, rows_v1, wgsem, wssem0, wssem1):
  c = lax.axis_index("c")
  s = lax.axis_index("s")
  wid = s * 2 + c
  nchunk_ns = (nchunk_n0, nchunk_n1)
  rows_vs = (rows_v0, rows_v1)
  wssems = (wssem0, wssem1)
  iota = lax.iota(jnp.int32, 16)

  pltpu.sync_copy(cnt_hbm.at[wid], cnt_v)
  pltpu.sync_copy(nodes_hbm.at[wid], nodes_v)
  lane0 = (iota == 0).astype(jnp.int32)
  nwin = jnp.sum(cnt_v[...] * lane0)

  nchunks = (nwin + CH - 1) // CH

  def chunk_body(ci, carry):
    off = ci * CH
    for par in range(2):
      @pl.when(lax.rem(ci, 2) == par)
      def _():
        nb = nchunk_ns[par]
        rb = rows_vs[par]

        @pl.when(ci >= 2)
        def _():
          pltpu.make_async_copy(rb, out_hbm.at[nb], wssems[par]).wait()

        for k in range(CH // 16):
          nb[pl.ds(k * 16, 16)] = nodes_v[pl.ds(off + k * 16, 16)]
        pltpu.async_copy(wbuf_hbm.at[wid].at[pl.ds(off, CH)], rb,
                         wgsem).wait()
        pltpu.make_async_copy(rb, out_hbm.at[nb], wssems[par]).start()
    return carry

  lax.fori_loop(0, nchunks, chunk_body, 0)

  @pl.when(nchunks >= 1)
  def _():
    par = lax.rem(nchunks - 1, 2)
    for p in range(2):
      @pl.when(par == p)
      def _():
        pltpu.make_async_copy(rows_vs[p], out_hbm.at[nchunk_ns[p]],
                              wssems[p]).wait()

  @pl.when(nchunks >= 2)
  def _():
    par = lax.rem(nchunks - 2, 2)
    for p in range(2):
      @pl.when(par == p)
      def _():
        pltpu.make_async_copy(rows_vs[p], out_hbm.at[nchunk_ns[p]],
                              wssems[p]).wait()


_sc_mesh = plsc.VectorSubcoreMesh(core_axis_name="c", subcore_axis_name="s")
_sc_params = pltpu.CompilerParams(needs_layout_passes=False)

_stage_a = functools.partial(
    pl.kernel,
    out_type=(
        jax.ShapeDtypeStruct((NW, WB), jnp.int32),
        jax.ShapeDtypeStruct((NW, 16), jnp.int32),
        jax.ShapeDtypeStruct((NW, WB, D), jnp.float32),
    ),
    mesh=_sc_mesh,
    compiler_params=_sc_params,
    scratch_types=[
        pltpu.VMEM((B,), jnp.int32),
        pltpu.VMEM((T,), jnp.int32),
        pltpu.VMEM((WB,), jnp.int32),
        pltpu.VMEM((WB,), jnp.int32),
        pltpu.VMEM((16,), jnp.int32),
        pltpu.VMEM((CH, D), jnp.float32),
        pltpu.VMEM((CH, D), jnp.float32),
        pltpu.SemaphoreType.DMA,
        pltpu.SemaphoreType.DMA,
        pltpu.SemaphoreType.DMA,
    ],
)(_body_a)

_stage_b = functools.partial(
    pl.kernel,
    out_type=(),
    mesh=_sc_mesh,
    compiler_params=_sc_params,
    scratch_types=[
        pltpu.VMEM((16,), jnp.int32),
        pltpu.VMEM((WB,), jnp.int32),
        pltpu.VMEM((CH,), jnp.int32),
        pltpu.VMEM((CH,), jnp.int32),
        pltpu.VMEM((CH, D), jnp.float32),
        pltpu.VMEM((CH, D), jnp.float32),
        pltpu.SemaphoreType.DMA,
        pltpu.SemaphoreType.DMA,
        pltpu.SemaphoreType.DMA,
    ],
)(_body_b)


def kernel(mem, idx, val):
  idx32 = idx.astype(jnp.int32)
  nodes, cnt, wbuf = _stage_a(idx32, val)
  out_ref = jax.new_ref(mem)
  _stage_b(nodes, cnt, wbuf, out_ref)
  return out_ref[...]


# speedup vs baseline: 1.1555x; 1.1555x over previous
import functools

import jax
import jax.numpy as jnp
from jax import lax
from jax.experimental import pallas as pl
from jax.experimental.pallas import tpu as pltpu
from jax.experimental.pallas import tpu_sc as plsc

M = 100000
B = 16384
D = 128
NW = 32
S = 3136
S_LAST = M - S * (NW - 1)
T = S
CH = 128
WB = S + CH
NCHUNK = B // 16
DPS = 64
BF = 8


def _dedup_chunks(idx_v, table_v, row_lo, n_own_u, iota, base, chunks):
  for group_start in range(0, len(chunks), BF):
    group = chunks[group_start:group_start + BF]
    ivecs = [idx_v[pl.ds((base + c) * 16, 16)] for c in group]
    locals_ = [ivec - row_lo for ivec in ivecs]
    valids = [l.astype(jnp.uint32) < n_own_u for l in locals_]
    lasts = [plsc.scan_count(ivec, mask=v)[1]
             for ivec, v in zip(ivecs, valids)]
    for cc, l, v, last in zip(group, locals_, valids, lasts):
      m = v & last
      l_c = jnp.where(m, l, 0)
      plsc.store_scatter(table_v, [l_c], (base + cc) * 16 + iota, mask=m)


def _body(idx_hbm, val_hbm, out_hbm, idx_v, table_v, nodes_v, gidx_v,
          nchunk_n0, nchunk_n1, rows_v0, rows_v1, wgsem, wssem0, wssem1):
  c = lax.axis_index("c")
  s = lax.axis_index("s")
  wid = s * 2 + c
  row_lo = wid * S
  n_own = jnp.where(wid == NW - 1, S_LAST, S)
  n_own_u = n_own.astype(jnp.uint32)
  nchunk_ns = (nchunk_n0, nchunk_n1)
  rows_vs = (rows_v0, rows_v1)
  wssems = (wssem0, wssem1)
  iota = lax.iota(jnp.int32, 16)

  pltpu.sync_copy(idx_hbm, idx_v)

  minus1 = jnp.full((16,), -1, jnp.int32)

  def zero_body(i, carry):
    for u in range(4):
      table_v[pl.ds((i * 4 + u) * 16, 16)] = minus1
    return carry

  lax.fori_loop(0, T // 16 // 4, zero_body, 0)

  def scan_body(i, carry):
    _dedup_chunks(idx_v, table_v, row_lo, n_own_u, iota, i * DPS,
                  list(range(DPS)))
    return carry

  lax.fori_loop(0, NCHUNK // DPS, scan_body, 0)

  def extract_body(t, off):
    tv = table_v[pl.ds(t * 16, 16)]
    m = tv >= 0
    nodes = (row_lo + t * 16) + iota
    plsc.store_compressed(nodes_v.at[pl.ds(off, 16)], nodes, mask=m)
    plsc.store_compressed(gidx_v.at[pl.ds(off, 16)], tv, mask=m)
    return off + jnp.sum(m.astype(jnp.int32))

  nwin = lax.fori_loop(0, T // 16, extract_body, jnp.int32(0))

  @pl.when(nwin > 0)
  def _():
    lane0 = (iota == 0).astype(jnp.int32)
    n0 = jnp.sum(nodes_v[pl.ds(0, 16)] * lane0)
    g0 = jnp.sum(gidx_v[pl.ds(0, 16)] * lane0)
    npad = jnp.zeros((16,), jnp.int32) + n0
    gpad = jnp.zeros((16,), jnp.int32) + g0
    for k in range(CH // 16):
      nodes_v[pl.ds(nwin + k * 16, 16)] = npad
      gidx_v[pl.ds(nwin + k * 16, 16)] = gpad

  nchunks = (nwin + CH - 1) // CH

  def chunk_body(ci, carry):
    off = ci * CH
    for par in range(2):
      @pl.when(lax.rem(ci, 2) == par)
      def _():
        nb = nchunk_ns[par]
        rb = rows_vs[par]

        @pl.when(ci >= 2)
        def _():
          pltpu.make_async_copy(rb, out_hbm.at[nb], wssems[par]).wait()

        for k in range(CH // 16):
          nb[pl.ds(k * 16, 16)] = nodes_v[pl.ds(off + k * 16, 16)]
        pltpu.async_copy(val_hbm.at[gidx_v.at[pl.ds(off, CH)]], rb,
                         wgsem).wait()
        pltpu.make_async_copy(rb, out_hbm.at[nb], wssems[par]).start()
    return carry

  lax.fori_loop(0, nchunks, chunk_body, 0)

  @pl.when(nchunks >= 1)
  def _():
    par = lax.rem(nchunks - 1, 2)
    for p in range(2):
      @pl.when(par == p)
      def _():
        pltpu.make_async_copy(rows_vs[p], out_hbm.at[nchunk_ns[p]],
                              wssems[p]).wait()

  @pl.when(nchunks >= 2)
  def _():
    par = lax.rem(nchunks - 2, 2)
    for p in range(2):
      @pl.when(par == p)
      def _():
        pltpu.make_async_copy(rows_vs[p], out_hbm.at[nchunk_ns[p]],
                              wssems[p]).wait()


_agg = functools.partial(
    pl.kernel,
    out_type=(),
    mesh=plsc.VectorSubcoreMesh(core_axis_name="c", subcore_axis_name="s"),
    compiler_params=pltpu.CompilerParams(needs_layout_passes=False),
    scratch_types=[
        pltpu.VMEM((B,), jnp.int32),
        pltpu.VMEM((T,), jnp.int32),
        pltpu.VMEM((WB,), jnp.int32),
        pltpu.VMEM((WB,), jnp.int32),
        pltpu.VMEM((CH,), jnp.int32),
        pltpu.VMEM((CH,), jnp.int32),
        pltpu.VMEM((CH, D), jnp.float32),
        pltpu.VMEM((CH, D), jnp.float32),
        pltpu.SemaphoreType.DMA,
        pltpu.SemaphoreType.DMA,
        pltpu.SemaphoreType.DMA,
    ],
)(_body)


def kernel(mem, idx, val):
  idx32 = idx.astype(jnp.int32)
  out_ref = jax.new_ref(mem)
  _agg(idx32, val, out_ref)
  return out_ref[...]


# speedup vs baseline: 1.2128x; 1.0496x over previous
import functools

import jax
import jax.numpy as jnp
from jax import lax
from jax.experimental import pallas as pl
from jax.experimental.pallas import tpu as pltpu
from jax.experimental.pallas import tpu_sc as plsc

M = 100000
B = 16384
D = 128
NW = 32
S = 3136
S_LAST = M - S * (NW - 1)
T = S
CH = 64
WB = S + CH
NCHUNK = B // 16
DPS = 64
BF = 8


def _dedup_chunks(idx_v, table_v, row_lo, n_own_u, iota, base, chunks):
  for group_start in range(0, len(chunks), BF):
    group = chunks[group_start:group_start + BF]
    ivecs = [idx_v[pl.ds((base + c) * 16, 16)] for c in group]
    locals_ = [ivec - row_lo for ivec in ivecs]
    valids = [l.astype(jnp.uint32) < n_own_u for l in locals_]
    lasts = [plsc.scan_count(ivec, mask=v)[1]
             for ivec, v in zip(ivecs, valids)]
    for cc, l, v, last in zip(group, locals_, valids, lasts):
      m = v & last
      l_c = jnp.where(m, l, 0)
      plsc.store_scatter(table_v, [l_c], (base + cc) * 16 + iota, mask=m)


def _body(idx_hbm, val_hbm, out_hbm, idx_v, table_v, nodes_v, gidx_v,
          nchunk_n0, nchunk_n1, rows_v0, rows_v1, wgsem, wssem0, wssem1):
  c = lax.axis_index("c")
  s = lax.axis_index("s")
  wid = s * 2 + c
  row_lo = wid * S
  n_own = jnp.where(wid == NW - 1, S_LAST, S)
  n_own_u = n_own.astype(jnp.uint32)
  nchunk_ns = (nchunk_n0, nchunk_n1)
  rows_vs = (rows_v0, rows_v1)
  wssems = (wssem0, wssem1)
  iota = lax.iota(jnp.int32, 16)

  pltpu.sync_copy(idx_hbm, idx_v)

  minus1 = jnp.full((16,), -1, jnp.int32)

  def zero_body(i, carry):
    for u in range(4):
      table_v[pl.ds((i * 4 + u) * 16, 16)] = minus1
    return carry

  lax.fori_loop(0, T // 16 // 4, zero_body, 0)

  def scan_body(i, carry):
    _dedup_chunks(idx_v, table_v, row_lo, n_own_u, iota, i * DPS,
                  list(range(DPS)))
    return carry

  lax.fori_loop(0, NCHUNK // DPS, scan_body, 0)

  def extract_body(t, off):
    tv = table_v[pl.ds(t * 16, 16)]
    m = tv >= 0
    nodes = (row_lo + t * 16) + iota
    plsc.store_compressed(nodes_v.at[pl.ds(off, 16)], nodes, mask=m)
    plsc.store_compressed(gidx_v.at[pl.ds(off, 16)], tv, mask=m)
    return off + jnp.sum(m.astype(jnp.int32))

  nwin = lax.fori_loop(0, T // 16, extract_body, jnp.int32(0))

  @pl.when(nwin > 0)
  def _():
    lane0 = (iota == 0).astype(jnp.int32)
    n0 = jnp.sum(nodes_v[pl.ds(0, 16)] * lane0)
    g0 = jnp.sum(gidx_v[pl.ds(0, 16)] * lane0)
    npad = jnp.zeros((16,), jnp.int32) + n0
    gpad = jnp.zeros((16,), jnp.int32) + g0
    for k in range(CH // 16):
      nodes_v[pl.ds(nwin + k * 16, 16)] = npad
      gidx_v[pl.ds(nwin + k * 16, 16)] = gpad

  nchunks = (nwin + CH - 1) // CH

  def chunk_body(ci, carry):
    off = ci * CH
    for par in range(2):
      @pl.when(lax.rem(ci, 2) == par)
      def _():
        nb = nchunk_ns[par]
        rb = rows_vs[par]

        @pl.when(ci >= 2)
        def _():
          pltpu.make_async_copy(rb, out_hbm.at[nb], wssems[par]).wait()

        for k in range(CH // 16):
          nb[pl.ds(k * 16, 16)] = nodes_v[pl.ds(off + k * 16, 16)]
        pltpu.async_copy(val_hbm.at[gidx_v.at[pl.ds(off, CH)]], rb,
                         wgsem).wait()
        pltpu.make_async_copy(rb, out_hbm.at[nb], wssems[par]).start()
    return carry

  lax.fori_loop(0, nchunks, chunk_body, 0)

  @pl.when(nchunks >= 1)
  def _():
    par = lax.rem(nchunks - 1, 2)
    for p in range(2):
      @pl.when(par == p)
      def _():
        pltpu.make_async_copy(rows_vs[p], out_hbm.at[nchunk_ns[p]],
                              wssems[p]).wait()

  @pl.when(nchunks >= 2)
  def _():
    par = lax.rem(nchunks - 2, 2)
    for p in range(2):
      @pl.when(par == p)
      def _():
        pltpu.make_async_copy(rows_vs[p], out_hbm.at[nchunk_ns[p]],
                              wssems[p]).wait()


_agg = functools.partial(
    pl.kernel,
    out_type=(),
    mesh=plsc.VectorSubcoreMesh(core_axis_name="c", subcore_axis_name="s"),
    compiler_params=pltpu.CompilerParams(needs_layout_passes=False),
    scratch_types=[
        pltpu.VMEM((B,), jnp.int32),
        pltpu.VMEM((T,), jnp.int32),
        pltpu.VMEM((WB,), jnp.int32),
        pltpu.VMEM((WB,), jnp.int32),
        pltpu.VMEM((CH,), jnp.int32),
        pltpu.VMEM((CH,), jnp.int32),
        pltpu.VMEM((CH, D), jnp.float32),
        pltpu.VMEM((CH, D), jnp.float32),
        pltpu.SemaphoreType.DMA,
        pltpu.SemaphoreType.DMA,
        pltpu.SemaphoreType.DMA,
    ],
)(_body)


def kernel(mem, idx, val):
  idx32 = idx.astype(jnp.int32)
  out_ref = jax.new_ref(mem)
  _agg(idx32, val, out_ref)
  return out_ref[...]


# speedup vs baseline: 1.2191x; 1.0052x over previous
import functools

import jax
import jax.numpy as jnp
from jax import lax
from jax.experimental import pallas as pl
from jax.experimental.pallas import tpu as pltpu
from jax.experimental.pallas import tpu_sc as plsc

M = 100000
B = 16384
D = 128
NW = 32
S = 3136
S_LAST = M - S * (NW - 1)
T = S
CH = 64
WB = S + CH
NCHUNK = B // 16
DPS = 64
BF = 16


def _dedup_chunks(idx_v, table_v, row_lo, n_own_u, iota, base, chunks):
  for group_start in range(0, len(chunks), BF):
    group = chunks[group_start:group_start + BF]
    ivecs = [idx_v[pl.ds((base + c) * 16, 16)] for c in group]
    locals_ = [ivec - row_lo for ivec in ivecs]
    valids = [l.astype(jnp.uint32) < n_own_u for l in locals_]
    lasts = [plsc.scan_count(ivec, mask=v)[1]
             for ivec, v in zip(ivecs, valids)]
    for cc, l, v, last in zip(group, locals_, valids, lasts):
      m = v & last
      l_c = jnp.where(m, l, 0)
      plsc.store_scatter(table_v, [l_c], (base + cc) * 16 + iota, mask=m)


def _body(idx_hbm, val_hbm, out_hbm, idx_v, table_v, nodes_v, gidx_v,
          nchunk_n0, nchunk_n1, rows_v0, rows_v1, wgsem, wssem0, wssem1):
  c = lax.axis_index("c")
  s = lax.axis_index("s")
  wid = s * 2 + c
  row_lo = wid * S
  n_own = jnp.where(wid == NW - 1, S_LAST, S)
  n_own_u = n_own.astype(jnp.uint32)
  nchunk_ns = (nchunk_n0, nchunk_n1)
  rows_vs = (rows_v0, rows_v1)
  wssems = (wssem0, wssem1)
  iota = lax.iota(jnp.int32, 16)

  pltpu.sync_copy(idx_hbm, idx_v)

  minus1 = jnp.full((16,), -1, jnp.int32)

  def zero_body(i, carry):
    for u in range(4):
      table_v[pl.ds((i * 4 + u) * 16, 16)] = minus1
    return carry

  lax.fori_loop(0, T // 16 // 4, zero_body, 0)

  def scan_body(i, carry):
    _dedup_chunks(idx_v, table_v, row_lo, n_own_u, iota, i * DPS,
                  list(range(DPS)))
    return carry

  lax.fori_loop(0, NCHUNK // DPS, scan_body, 0)

  def extract_body(t, off):
    tv = table_v[pl.ds(t * 16, 16)]
    m = tv >= 0
    nodes = (row_lo + t * 16) + iota
    plsc.store_compressed(nodes_v.at[pl.ds(off, 16)], nodes, mask=m)
    plsc.store_compressed(gidx_v.at[pl.ds(off, 16)], tv, mask=m)
    return off + jnp.sum(m.astype(jnp.int32))

  nwin = lax.fori_loop(0, T // 16, extract_body, jnp.int32(0))

  @pl.when(nwin > 0)
  def _():
    lane0 = (iota == 0).astype(jnp.int32)
    n0 = jnp.sum(nodes_v[pl.ds(0, 16)] * lane0)
    g0 = jnp.sum(gidx_v[pl.ds(0, 16)] * lane0)
    npad = jnp.zeros((16,), jnp.int32) + n0
    gpad = jnp.zeros((16,), jnp.int32) + g0
    for k in range(CH // 16):
      nodes_v[pl.ds(nwin + k * 16, 16)] = npad
      gidx_v[pl.ds(nwin + k * 16, 16)] = gpad

  nchunks = (nwin + CH - 1) // CH

  def chunk_body(ci, carry):
    off = ci * CH
    for par in range(2):
      @pl.when(lax.rem(ci, 2) == par)
      def _():
        nb = nchunk_ns[par]
        rb = rows_vs[par]

        @pl.when(ci >= 2)
        def _():
          pltpu.make_async_copy(rb, out_hbm.at[nb], wssems[par]).wait()

        for k in range(CH // 16):
          nb[pl.ds(k * 16, 16)] = nodes_v[pl.ds(off + k * 16, 16)]
        pltpu.async_copy(val_hbm.at[gidx_v.at[pl.ds(off, CH)]], rb,
                         wgsem).wait()
        pltpu.make_async_copy(rb, out_hbm.at[nb], wssems[par]).start()
    return carry

  lax.fori_loop(0, nchunks, chunk_body, 0)

  @pl.when(nchunks >= 1)
  def _():
    par = lax.rem(nchunks - 1, 2)
    for p in range(2):
      @pl.when(par == p)
      def _():
        pltpu.make_async_copy(rows_vs[p], out_hbm.at[nchunk_ns[p]],
                              wssems[p]).wait()

  @pl.when(nchunks >= 2)
  def _():
    par = lax.rem(nchunks - 2, 2)
    for p in range(2):
      @pl.when(par == p)
      def _():
        pltpu.make_async_copy(rows_vs[p], out_hbm.at[nchunk_ns[p]],
                              wssems[p]).wait()


_agg = functools.partial(
    pl.kernel,
    out_type=(),
    mesh=plsc.VectorSubcoreMesh(core_axis_name="c", subcore_axis_name="s"),
    compiler_params=pltpu.CompilerParams(needs_layout_passes=False),
    scratch_types=[
        pltpu.VMEM((B,), jnp.int32),
        pltpu.VMEM((T,), jnp.int32),
        pltpu.VMEM((WB,), jnp.int32),
        pltpu.VMEM((WB,), jnp.int32),
        pltpu.VMEM((CH,), jnp.int32),
        pltpu.VMEM((CH,), jnp.int32),
        pltpu.VMEM((CH, D), jnp.float32),
        pltpu.VMEM((CH, D), jnp.float32),
        pltpu.SemaphoreType.DMA,
        pltpu.SemaphoreType.DMA,
        pltpu.SemaphoreType.DMA,
    ],
)(_body)


def kernel(mem, idx, val):
  idx32 = idx.astype(jnp.int32)
  out_ref = jax.new_ref(mem)
  _agg(idx32, val, out_ref)
  return out_ref[...]


# speedup vs baseline: 1.2743x; 1.0453x over previous
import functools

import jax
import jax.numpy as jnp
from jax import lax
from jax.experimental import pallas as pl
from jax.experimental.pallas import tpu as pltpu
from jax.experimental.pallas import tpu_sc as plsc

M = 100000
B = 16384
D = 128
NW = 32
S = 3136
S_LAST = M - S * (NW - 1)
T = S
CH = 64
WB = S + CH
NCHUNK = B // 16
DPS = 64
BF = 16


def _dedup_chunks(idx_v, table_v, row_lo, n_own_u, iota, base, chunks):
  for group_start in range(0, len(chunks), BF):
    group = chunks[group_start:group_start + BF]
    ivecs = [idx_v[pl.ds((base + c) * 16, 16)] for c in group]
    locals_ = [ivec - row_lo for ivec in ivecs]
    valids = [l.astype(jnp.uint32) < n_own_u for l in locals_]
    lasts = [plsc.scan_count(ivec, mask=v)[1]
             for ivec, v in zip(ivecs, valids)]
    for cc, l, v, last in zip(group, locals_, valids, lasts):
      m = v & last
      l_c = jnp.where(m, l, 0)
      plsc.store_scatter(table_v, [l_c], (base + cc) * 16 + iota, mask=m)


def _body(idx_hbm, val_hbm, out_hbm, idx_v, table_v, nodes_v, gidx_v,
          nchunk_n0, nchunk_n1, rows_v0, rows_v1, wgsem0, wgsem1, wssem0,
          wssem1):
  c = lax.axis_index("c")
  s = lax.axis_index("s")
  wid = s * 2 + c
  row_lo = wid * S
  n_own = jnp.where(wid == NW - 1, S_LAST, S)
  n_own_u = n_own.astype(jnp.uint32)
  nchunk_ns = (nchunk_n0, nchunk_n1)
  rows_vs = (rows_v0, rows_v1)
  wssems = (wssem0, wssem1)
  iota = lax.iota(jnp.int32, 16)

  pltpu.sync_copy(idx_hbm, idx_v)

  minus1 = jnp.full((16,), -1, jnp.int32)

  def zero_body(i, carry):
    for u in range(4):
      table_v[pl.ds((i * 4 + u) * 16, 16)] = minus1
    return carry

  lax.fori_loop(0, T // 16 // 4, zero_body, 0)

  def scan_body(i, carry):
    _dedup_chunks(idx_v, table_v, row_lo, n_own_u, iota, i * DPS,
                  list(range(DPS)))
    return carry

  lax.fori_loop(0, NCHUNK // DPS, scan_body, 0)

  def extract_body(t, off):
    tv = table_v[pl.ds(t * 16, 16)]
    m = tv >= 0
    nodes = (row_lo + t * 16) + iota
    plsc.store_compressed(nodes_v.at[pl.ds(off, 16)], nodes, mask=m)
    plsc.store_compressed(gidx_v.at[pl.ds(off, 16)], tv, mask=m)
    return off + jnp.sum(m.astype(jnp.int32))

  nwin = lax.fori_loop(0, T // 16, extract_body, jnp.int32(0))

  @pl.when(nwin > 0)
  def _():
    lane0 = (iota == 0).astype(jnp.int32)
    n0 = jnp.sum(nodes_v[pl.ds(0, 16)] * lane0)
    g0 = jnp.sum(gidx_v[pl.ds(0, 16)] * lane0)
    npad = jnp.zeros((16,), jnp.int32) + n0
    gpad = jnp.zeros((16,), jnp.int32) + g0
    for k in range(CH // 16):
      nodes_v[pl.ds(nwin + k * 16, 16)] = npad
      gidx_v[pl.ds(nwin + k * 16, 16)] = gpad

  nchunks = (nwin + CH - 1) // CH
  gsems = (wgsem0, wgsem1)

  def gather_cp(par, off):
    return pltpu.make_async_copy(
        val_hbm.at[gidx_v.at[pl.ds(off, CH)]], rows_vs[par], gsems[par])

  @pl.when(nchunks >= 1)
  def _():
    gather_cp(0, 0).start()

  def chunk_body(ci, carry):
    off = ci * CH
    for par in range(2):
      @pl.when(lax.rem(ci, 2) == par)
      def _():
        opar = 1 - par
        nb = nchunk_ns[par]
        rb = rows_vs[par]

        @pl.when(ci >= 1)
        def _():
          pltpu.make_async_copy(rows_vs[opar], out_hbm.at[nchunk_ns[opar]],
                                wssems[opar]).wait()

        @pl.when(ci + 1 < nchunks)
        def _():
          gather_cp(opar, off + CH).start()

        gather_cp(par, off).wait()
        for k in range(CH // 16):
          nb[pl.ds(k * 16, 16)] = nodes_v[pl.ds(off + k * 16, 16)]
        pltpu.make_async_copy(rb, out_hbm.at[nb], wssems[par]).start()
    return carry

  lax.fori_loop(0, nchunks, chunk_body, 0)

  @pl.when(nchunks >= 1)
  def _():
    par = lax.rem(nchunks - 1, 2)
    for p in range(2):
      @pl.when(par == p)
      def _():
        pltpu.make_async_copy(rows_vs[p], out_hbm.at[nchunk_ns[p]],
                              wssems[p]).wait()


_agg = functools.partial(
    pl.kernel,
    out_type=(),
    mesh=plsc.VectorSubcoreMesh(core_axis_name="c", subcore_axis_name="s"),
    compiler_params=pltpu.CompilerParams(needs_layout_passes=False),
    scratch_types=[
        pltpu.VMEM((B,), jnp.int32),
        pltpu.VMEM((T,), jnp.int32),
        pltpu.VMEM((WB,), jnp.int32),
        pltpu.VMEM((WB,), jnp.int32),
        pltpu.VMEM((CH,), jnp.int32),
        pltpu.VMEM((CH,), jnp.int32),
        pltpu.VMEM((CH, D), jnp.float32),
        pltpu.VMEM((CH, D), jnp.float32),
        pltpu.SemaphoreType.DMA,
        pltpu.SemaphoreType.DMA,
        pltpu.SemaphoreType.DMA,
        pltpu.SemaphoreType.DMA,
    ],
)(_body)


def kernel(mem, idx, val):
  idx32 = idx.astype(jnp.int32)
  out_ref = jax.new_ref(mem)
  _agg(idx32, val, out_ref)
  return out_ref[...]
